# baseline (speedup 1.0000x reference)
import jax
import jax.numpy as jnp
from jax.experimental import pallas as pl
from jax.experimental.pallas import tpu as pltpu

_N = 16384
_DIM = 768
_SD = 64
_H = 128
_K = 1024
_LR = 0.01
_BLK = 1024
_NBLK = _N // _BLK


def _body(x_ref, w1t_ref, b1_ref, w2tp_ref, b2p_ref, st_ref, schemas_ref,
          usage_ref, ns_ref, nu_ref, cnt_ref, mn_ref, acc_ref, stn_ref):
    i = pl.program_id(0)

    @pl.when(i == 0)
    def _init():
        acc_ref[...] = jnp.zeros_like(acc_ref)
        stp = st_ref[...]
        n2sq = jnp.sum(stp * stp, axis=0, keepdims=True)
        invn2 = 1.0 / jnp.maximum(jnp.sqrt(n2sq), 1e-30)
        stn_ref[...] = stp * invn2

    x = x_ref[...]
    h = jnp.maximum(
        jnp.dot(x, w1t_ref[...], preferred_element_type=jnp.float32)
        + b1_ref[...], 0.0)
    ep = (jnp.dot(h, w2tp_ref[...], preferred_element_type=jnp.float32)
          + b2p_ref[...])

    sims = jnp.dot(ep, stn_ref[...],
                   preferred_element_type=jnp.float32)

    rowmax = jnp.max(sims, axis=1, keepdims=True)
    kiota = jax.lax.broadcasted_iota(jnp.int32, (_BLK, _K), 1)
    masked_idx = jnp.where(sims == rowmax, kiota, _K)
    amin = jnp.min(masked_idx, axis=1, keepdims=True)
    onehot = (masked_idx == amin).astype(jnp.bfloat16)

    acc_ref[...] += jax.lax.dot_general(
        onehot, ep.astype(jnp.bfloat16), (((0,), (0,)), ((), ())),
        preferred_element_type=jnp.float32)

    @pl.when(i == _NBLK - 1)
    def _finish():
        acc = acc_ref[...]
        lane_k = jax.lax.broadcasted_iota(jnp.int32, (_K, _H), 1)
        counts = jnp.sum(jnp.where(lane_k == _SD, acc, 0.0), axis=1,
                         keepdims=True)
        maxc = jnp.maximum(counts, 1.0)
        target = acc / maxc
        active = counts > 0.0
        delta = jnp.where(jnp.logical_and(active, lane_k < _SD),
                          _LR * (target - schemas_ref[...]), 0.0)
        ns_ref[...] = schemas_ref[...] + delta
        nu_ref[...] = usage_ref[...] + counts
        nrm = jnp.sqrt(jnp.sum(delta * delta, axis=1, keepdims=True))
        activef = active.astype(jnp.float32)
        num_up = jnp.sum(activef, axis=0, keepdims=True)
        cnt_ref[...] = num_up.astype(jnp.int32)
        mn_ref[...] = (jnp.sum(jnp.where(active, nrm, 0.0), axis=0,
                               keepdims=True)
                       / jnp.maximum(num_up, 1.0))


def kernel(episodic_traces, W1, b1, W2, b2, schemas, schema_usage):
    f32 = jnp.float32
    w1t = W1.T
    w2tp = jnp.zeros((_H, _H), f32).at[:, :_SD].set(W2.T)
    b2p = jnp.zeros((1, _H), f32).at[0, :_SD].set(b2).at[0, _SD].set(1.0)
    st_pad = jnp.zeros((_H, _K), f32).at[:_SD, :].set(schemas.T)
    schemas_pad = jnp.zeros((_K, _H), f32).at[:, :_SD].set(schemas)
    usage2 = schema_usage[:, None]

    const = lambda *_: (0, 0)
    grid = (_NBLK,)
    out = pl.pallas_call(
        _body,
        grid=grid,
        in_specs=[
            pl.BlockSpec((_BLK, _DIM), lambda i: (i, 0)),
            pl.BlockSpec((_DIM, _H), const),
            pl.BlockSpec((1, _H), const),
            pl.BlockSpec((_H, _H), const),
            pl.BlockSpec((1, _H), const),
            pl.BlockSpec((_H, _K), const),
            pl.BlockSpec((_K, _H), const),
            pl.BlockSpec((_K, 1), const),
        ],
        out_specs=[
            pl.BlockSpec((_K, _H), const),
            pl.BlockSpec((_K, 1), const),
            pl.BlockSpec((1, 1), const),
            pl.BlockSpec((1, 1), const),
        ],
        out_shape=[
            jax.ShapeDtypeStruct((_K, _H), f32),
            jax.ShapeDtypeStruct((_K, 1), f32),
            jax.ShapeDtypeStruct((1, 1), jnp.int32),
            jax.ShapeDtypeStruct((1, 1), f32),
        ],
        scratch_shapes=[pltpu.VMEM((_K, _H), f32),
                        pltpu.VMEM((_H, _K), f32)],
    )(episodic_traces, w1t, b1[None, :], w2tp, b2p, st_pad, schemas_pad,
      usage2)
    ns_pad, nu2, cnt, mn = out
    return (ns_pad[:, :_SD], nu2[:, 0], cnt[0, 0], mn[0, 0])


# speedup vs baseline: 1.0565x; 1.0565x over previous
import jax
import jax.numpy as jnp
from jax.experimental import pallas as pl
from jax.experimental.pallas import tpu as pltpu

_N = 16384
_DIM = 768
_SD = 64
_H = 128
_K = 1024
_LR = 0.01
_BLK = 1024
_NBLK = _N // _BLK


def _body(x_ref, w1t_ref, b1_ref, w2tp_ref, b2p_ref, st_ref, schemas_ref,
          usage_ref, ns_ref, nu_ref, cnt_ref, mn_ref, acc_ref):
    i = pl.program_id(0)

    @pl.when(i == 0)
    def _init():
        acc_ref[...] = jnp.zeros_like(acc_ref)

    x = x_ref[...]
    h = jnp.maximum(
        jnp.dot(x, w1t_ref[...], preferred_element_type=jnp.float32)
        + b1_ref[...], 0.0)
    ep = (jnp.dot(h, w2tp_ref[...], preferred_element_type=jnp.float32)
          + b2p_ref[...])

    stp = st_ref[...]
    n2sq = jnp.sum(stp * stp, axis=0, keepdims=True)
    invn2 = 1.0 / jnp.maximum(jnp.sqrt(n2sq), 1e-30)
    dot = jnp.dot(ep, stp, preferred_element_type=jnp.float32)
    sims = dot * invn2

    rowmax = jnp.max(sims, axis=1, keepdims=True)
    kiota = jax.lax.broadcasted_iota(jnp.int32, (_BLK, _K), 1)
    masked_idx = jnp.where(sims == rowmax, kiota, _K)
    amin = jnp.min(masked_idx, axis=1, keepdims=True)
    onehot = (masked_idx == amin).astype(jnp.bfloat16)

    acc_ref[...] += jax.lax.dot_general(
        onehot, ep.astype(jnp.bfloat16), (((0,), (0,)), ((), ())),
        preferred_element_type=jnp.float32)

    @pl.when(i == _NBLK - 1)
    def _finish():
        acc = acc_ref[...]
        lane_k = jax.lax.broadcasted_iota(jnp.int32, (_K, _H), 1)
        counts = jnp.sum(jnp.where(lane_k == _SD, acc, 0.0), axis=1,
                         keepdims=True)
        maxc = jnp.maximum(counts, 1.0)
        target = acc / maxc
        active = counts > 0.0
        delta = jnp.where(jnp.logical_and(active, lane_k < _SD),
                          _LR * (target - schemas_ref[...]), 0.0)
        ns_ref[...] = schemas_ref[...] + delta
        nu_ref[...] = usage_ref[...] + counts
        nrm = jnp.sqrt(jnp.sum(delta * delta, axis=1, keepdims=True))
        activef = active.astype(jnp.float32)
        num_up = jnp.sum(activef, axis=0, keepdims=True)
        cnt_ref[...] = num_up.astype(jnp.int32)
        mn_ref[...] = (jnp.sum(jnp.where(active, nrm, 0.0), axis=0,
                               keepdims=True)
                       / jnp.maximum(num_up, 1.0))


def kernel(episodic_traces, W1, b1, W2, b2, schemas, schema_usage):
    f32 = jnp.float32
    w1t = W1.T
    w2tp = jnp.zeros((_H, _H), f32).at[:, :_SD].set(W2.T)
    b2p = jnp.zeros((1, _H), f32).at[0, :_SD].set(b2).at[0, _SD].set(1.0)
    st_pad = jnp.zeros((_H, _K), f32).at[:_SD, :].set(schemas.T)
    schemas_pad = jnp.zeros((_K, _H), f32).at[:, :_SD].set(schemas)
    usage2 = schema_usage[:, None]

    const = lambda *_: (0, 0)
    grid = (_NBLK,)
    out = pl.pallas_call(
        _body,
        grid=grid,
        in_specs=[
            pl.BlockSpec((_BLK, _DIM), lambda i: (i, 0)),
            pl.BlockSpec((_DIM, _H), const),
            pl.BlockSpec((1, _H), const),
            pl.BlockSpec((_H, _H), const),
            pl.BlockSpec((1, _H), const),
            pl.BlockSpec((_H, _K), const),
            pl.BlockSpec((_K, _H), const),
            pl.BlockSpec((_K, 1), const),
        ],
        out_specs=[
            pl.BlockSpec((_K, _H), const),
            pl.BlockSpec((_K, 1), const),
            pl.BlockSpec((1, 1), const),
            pl.BlockSpec((1, 1), const),
        ],
        out_shape=[
            jax.ShapeDtypeStruct((_K, _H), f32),
            jax.ShapeDtypeStruct((_K, 1), f32),
            jax.ShapeDtypeStruct((1, 1), jnp.int32),
            jax.ShapeDtypeStruct((1, 1), f32),
        ],
        scratch_shapes=[pltpu.VMEM((_K, _H), f32)],
    )(episodic_traces, w1t, b1[None, :], w2tp, b2p, st_pad, schemas_pad,
      usage2)
    ns_pad, nu2, cnt, mn = out
    return (ns_pad[:, :_SD], nu2[:, 0], cnt[0, 0], mn[0, 0])
